# baseline (speedup 1.0000x reference)
import functools

import jax
import jax.numpy as jnp
from jax import lax
from jax.experimental import pallas as pl

K = 30


def _emb_body(x_ref, w1_ref, b1_ref, w2_ref, b2_ref, out_ref):
    x = x_ref[...]
    h = lax.dot_general(x, w1_ref[...], (((1,), (1,)), ((), ())),
                        preferred_element_type=jnp.float32)
    h = jnp.maximum(h + b1_ref[...], 0.0)
    h = lax.dot_general(h, w2_ref[...], (((1,), (1,)), ((), ())),
                        preferred_element_type=jnp.float32)
    h = h + b2_ref[...]
    norm = jnp.sqrt(jnp.sum(h * h, axis=1, keepdims=True))
    out_ref[...] = h / jnp.maximum(norm, 1e-12)


def _sim_body(rows_ref, emb_ref, out_ref, *, kk):
    rows = rows_ref[...]
    emb = emb_ref[...]
    s = lax.dot_general(rows, emb, (((1,), (1,)), ((), ())),
                        preferred_element_type=jnp.float32)
    st = lax.dot_general(emb, rows, (((1,), (1,)), ((), ())),
                         preferred_element_type=jnp.float32)
    br = s.shape[0]
    n = s.shape[1]
    ninf = jnp.float32(-jnp.inf)

    nplanes = n // 256
    t1 = t2 = t3 = t4 = jnp.full((256, br), ninf, dtype=jnp.float32)
    for a in range(nplanes):
        v = st[a * 256:(a + 1) * 256, :]
        n1 = jnp.maximum(t1, v)
        v = jnp.minimum(t1, v)
        n2 = jnp.maximum(t2, v)
        v = jnp.minimum(t2, v)
        n3 = jnp.maximum(t3, v)
        v = jnp.minimum(t3, v)
        n4 = jnp.maximum(t4, v)
        t1, t2, t3, t4 = n1, n2, n3, n4

    def pop(_, carry):
        t1, t2, t3, t4, _ = carry
        m = jnp.max(t1, axis=0, keepdims=True)
        upd = t1 == m
        return (jnp.where(upd, t2, t1), jnp.where(upd, t3, t2),
                jnp.where(upd, t4, t3), jnp.where(upd, ninf, t4), m)

    m0 = jnp.full((1, br), jnp.inf, dtype=jnp.float32)
    tau_t = lax.fori_loop(0, kk, pop, (t1, t2, t3, t4, m0))[4]
    tau = tau_t.reshape(br, 1)
    out_ref[...] = jnp.where(s >= tau, jnp.maximum(s, 0.0), 0.0)


def kernel(features, W1, b1, W2, b2):
    n, d = features.shape
    emb = pl.pallas_call(
        _emb_body,
        out_shape=jax.ShapeDtypeStruct((n, d), jnp.float32),
    )(features, W1, b1.reshape(1, d), W2, b2.reshape(1, d))

    br = 128
    grid = (n // br,)
    out = pl.pallas_call(
        functools.partial(_sim_body, kk=K + 1),
        grid=grid,
        in_specs=[
            pl.BlockSpec((br, d), lambda i: (i, 0)),
            pl.BlockSpec((n, d), lambda i: (0, 0)),
        ],
        out_specs=pl.BlockSpec((br, n), lambda i: (i, 0)),
        out_shape=jax.ShapeDtypeStruct((n, n), jnp.float32),
    )(emb, emb)
    return out


# speedup vs baseline: 1.7077x; 1.7077x over previous
import functools

import jax
import jax.numpy as jnp
from jax import lax
from jax.experimental import pallas as pl

K = 30


def _emb_body(x_ref, w1_ref, b1_ref, w2_ref, b2_ref, out_ref):
    x = x_ref[...]
    h = lax.dot_general(x, w1_ref[...], (((1,), (1,)), ((), ())),
                        preferred_element_type=jnp.float32)
    h = jnp.maximum(h + b1_ref[...], 0.0)
    h = lax.dot_general(h, w2_ref[...], (((1,), (1,)), ((), ())),
                        preferred_element_type=jnp.float32)
    h = h + b2_ref[...]
    norm = jnp.sqrt(jnp.sum(h * h, axis=1, keepdims=True))
    out_ref[...] = h / jnp.maximum(norm, 1e-12)


def _sim_body(rows_ref, emb_ref, out_ref, *, kk):
    rows = rows_ref[...]
    emb = emb_ref[...]
    s = lax.dot_general(rows, emb, (((1,), (1,)), ((), ())),
                        preferred_element_type=jnp.float32)
    st = lax.dot_general(emb, rows, (((1,), (1,)), ((), ())),
                         preferred_element_type=jnp.float32)
    br = s.shape[0]
    n = s.shape[1]
    ninf = jnp.float32(-jnp.inf)

    nc, nd = 64, 6
    stacks = [jnp.full((nc, br), ninf, dtype=jnp.float32)] * nd
    for a in range(n // nc):
        v = st[a * nc:(a + 1) * nc, :]
        new = []
        for i in range(nd):
            new.append(jnp.maximum(stacks[i], v))
            if i < nd - 1:
                v = jnp.minimum(stacks[i], v)
        stacks = new

    m = None
    for _ in range(kk):
        m = jnp.max(stacks[0], axis=0, keepdims=True)
        upd = stacks[0] == m
        stacks = ([jnp.where(upd, stacks[i + 1], stacks[i])
                   for i in range(nd - 1)]
                  + [jnp.where(upd, ninf, stacks[nd - 1])])
    tau = m.reshape(br, 1)
    out_ref[...] = jnp.where(s >= tau, jnp.maximum(s, 0.0), 0.0)


def kernel(features, W1, b1, W2, b2):
    n, d = features.shape
    emb = pl.pallas_call(
        _emb_body,
        out_shape=jax.ShapeDtypeStruct((n, d), jnp.float32),
    )(features, W1, b1.reshape(1, d), W2, b2.reshape(1, d))

    br = 128
    grid = (n // br,)
    out = pl.pallas_call(
        functools.partial(_sim_body, kk=K + 1),
        grid=grid,
        in_specs=[
            pl.BlockSpec((br, d), lambda i: (i, 0)),
            pl.BlockSpec((n, d), lambda i: (0, 0)),
        ],
        out_specs=pl.BlockSpec((br, n), lambda i: (i, 0)),
        out_shape=jax.ShapeDtypeStruct((n, n), jnp.float32),
    )(emb, emb)
    return out


# speedup vs baseline: 1.7504x; 1.0250x over previous
import functools

import jax
import jax.numpy as jnp
from jax import lax
from jax.experimental import pallas as pl

K = 30


def _emb_body(x_ref, w1_ref, b1_ref, w2_ref, b2_ref, out_ref):
    x = x_ref[...]
    h = lax.dot_general(x, w1_ref[...], (((1,), (1,)), ((), ())),
                        preferred_element_type=jnp.float32)
    h = jnp.maximum(h + b1_ref[...], 0.0)
    h = lax.dot_general(h, w2_ref[...], (((1,), (1,)), ((), ())),
                        preferred_element_type=jnp.float32)
    h = h + b2_ref[...]
    norm = jnp.sqrt(jnp.sum(h * h, axis=1, keepdims=True))
    out_ref[...] = h / jnp.maximum(norm, 1e-12)


def _sim_body(rows_ref, emb_ref, out_ref, *, kk):
    rows = rows_ref[...]
    emb = emb_ref[...]
    s = lax.dot_general(rows, emb, (((1,), (1,)), ((), ())),
                        preferred_element_type=jnp.float32)
    st = lax.dot_general(emb, rows, (((1,), (1,)), ((), ())),
                         preferred_element_type=jnp.float32)
    br = s.shape[0]
    n = s.shape[1]
    ninf = jnp.float32(-jnp.inf)

    nc, nd = 64, 6
    stacks = [jnp.full((nc, br), ninf, dtype=jnp.float32)] * nd
    for a in range(n // nc):
        v = st[a * nc:(a + 1) * nc, :]
        new = []
        for i in range(nd):
            new.append(jnp.maximum(stacks[i], v))
            if i < nd - 1:
                v = jnp.minimum(stacks[i], v)
        stacks = new

    m = None
    for _ in range(kk):
        m = jnp.max(stacks[0], axis=0, keepdims=True)
        upd = stacks[0] == m
        stacks = ([jnp.where(upd, stacks[i + 1], stacks[i])
                   for i in range(nd - 1)]
                  + [jnp.where(upd, ninf, stacks[nd - 1])])
    tau = m.reshape(br, 1)
    out_ref[...] = jnp.where(s >= tau, jnp.maximum(s, 0.0), 0.0)


def kernel(features, W1, b1, W2, b2):
    n, d = features.shape
    emb = pl.pallas_call(
        _emb_body,
        out_shape=jax.ShapeDtypeStruct((n, d), jnp.float32),
    )(features, W1, b1.reshape(1, d), W2, b2.reshape(1, d))

    br = 256
    grid = (n // br,)
    out = pl.pallas_call(
        functools.partial(_sim_body, kk=K + 1),
        grid=grid,
        in_specs=[
            pl.BlockSpec((br, d), lambda i: (i, 0)),
            pl.BlockSpec((n, d), lambda i: (0, 0)),
        ],
        out_specs=pl.BlockSpec((br, n), lambda i: (i, 0)),
        out_shape=jax.ShapeDtypeStruct((n, n), jnp.float32),
    )(emb, emb)
    return out


# speedup vs baseline: 2.0528x; 1.1728x over previous
import functools

import jax
import jax.numpy as jnp
from jax import lax
from jax.experimental import pallas as pl

K = 30


def _emb_body(x_ref, w1_ref, b1_ref, w2_ref, b2_ref, out_ref):
    x = x_ref[...]
    h = lax.dot_general(x, w1_ref[...], (((1,), (1,)), ((), ())),
                        preferred_element_type=jnp.float32)
    h = jnp.maximum(h + b1_ref[...], 0.0)
    h = lax.dot_general(h, w2_ref[...], (((1,), (1,)), ((), ())),
                        preferred_element_type=jnp.float32)
    h = h + b2_ref[...]
    norm = jnp.sqrt(jnp.sum(h * h, axis=1, keepdims=True))
    out_ref[...] = h / jnp.maximum(norm, 1e-12)


def _sim_body(rows_ref, emb_ref, out_ref, *, kk):
    rows = rows_ref[...]
    emb = emb_ref[...]
    s = lax.dot_general(rows, emb, (((1,), (1,)), ((), ())),
                        preferred_element_type=jnp.float32)
    st = lax.dot_general(emb, rows, (((1,), (1,)), ((), ())),
                         preferred_element_type=jnp.float32)
    br = s.shape[0]
    n = s.shape[1]
    ninf = jnp.float32(-jnp.inf)

    nd = 6
    sub = [jnp.full((256, br), ninf, dtype=jnp.float32)] * 4
    for a in range(n // 256):
        v = st[a * 256:(a + 1) * 256, :]
        new = []
        for i in range(4):
            new.append(jnp.maximum(sub[i], v))
            if i < 3:
                v = jnp.minimum(sub[i], v)
        sub = new

    def ce(x, y):
        return jnp.maximum(x, y), jnp.minimum(x, y)

    def merge22(a, b):
        e0, e1 = ce(a[0], b[0])
        o0, o1 = ce(a[1], b[1])
        m1, m2 = ce(o0, e1)
        return [e0, m1, m2, o1]

    def merge44(a, b):
        e = merge22([a[0], a[2]], [b[0], b[2]])
        o = merge22([a[1], a[3]], [b[1], b[3]])
        out = [e[0]]
        for i in range(3):
            hi, lo = ce(o[i], e[i + 1])
            out += [hi, lo]
        out.append(o[3])
        return out

    sk = [[sub[i][k * 64:(k + 1) * 64, :] for i in range(4)]
          for k in range(4)]
    m12 = merge44(sk[0], sk[1])
    m34 = merge44(sk[2], sk[3])
    c = [jnp.maximum(m12[i], m34[7 - i]) for i in range(8)]
    for i in range(4):
        c[i], c[i + 4] = ce(c[i], c[i + 4])
    for i in (0, 1, 4, 5):
        c[i], c[i + 2] = ce(c[i], c[i + 2])
    for i in (0, 2, 4):
        c[i], c[i + 1] = ce(c[i], c[i + 1])
    stacks = c[:nd]

    m = None
    for _ in range(kk):
        m = jnp.max(stacks[0], axis=0, keepdims=True)
        upd = stacks[0] == m
        stacks = ([jnp.where(upd, stacks[i + 1], stacks[i])
                   for i in range(nd - 1)]
                  + [jnp.where(upd, ninf, stacks[nd - 1])])
    tau = m.reshape(br, 1)
    out_ref[...] = jnp.where(s >= tau, jnp.maximum(s, 0.0), 0.0)


def kernel(features, W1, b1, W2, b2):
    n, d = features.shape
    emb = pl.pallas_call(
        _emb_body,
        out_shape=jax.ShapeDtypeStruct((n, d), jnp.float32),
    )(features, W1, b1.reshape(1, d), W2, b2.reshape(1, d))

    br = 256
    grid = (n // br,)
    out = pl.pallas_call(
        functools.partial(_sim_body, kk=K + 1),
        grid=grid,
        in_specs=[
            pl.BlockSpec((br, d), lambda i: (i, 0)),
            pl.BlockSpec((n, d), lambda i: (0, 0)),
        ],
        out_specs=pl.BlockSpec((br, n), lambda i: (i, 0)),
        out_shape=jax.ShapeDtypeStruct((n, n), jnp.float32),
    )(emb, emb)
    return out


# speedup vs baseline: 2.1543x; 1.0494x over previous
import functools

import jax
import jax.numpy as jnp
from jax import lax
from jax.experimental import pallas as pl

K = 30


def _emb_body(x_ref, w1_ref, b1_ref, w2_ref, b2_ref, out_ref):
    x = x_ref[...]
    h = lax.dot_general(x, w1_ref[...], (((1,), (1,)), ((), ())),
                        preferred_element_type=jnp.float32)
    h = jnp.maximum(h + b1_ref[...], 0.0)
    h = lax.dot_general(h, w2_ref[...], (((1,), (1,)), ((), ())),
                        preferred_element_type=jnp.float32)
    h = h + b2_ref[...]
    norm = jnp.sqrt(jnp.sum(h * h, axis=1, keepdims=True))
    out_ref[...] = h / jnp.maximum(norm, 1e-12)


def _sim_body(rows_ref, emb_ref, out_ref, *, kk):
    rows = rows_ref[...]
    emb = emb_ref[...]
    s = lax.dot_general(rows, emb, (((1,), (1,)), ((), ())),
                        preferred_element_type=jnp.float32)
    st = lax.dot_general(emb, rows, (((1,), (1,)), ((), ())),
                         preferred_element_type=jnp.float32)
    br = s.shape[0]
    n = s.shape[1]
    ninf = jnp.float32(-jnp.inf)

    nd = 6
    sub = [jnp.full((256, br), ninf, dtype=jnp.float32)] * 4
    for a in range(n // 256):
        v = st[a * 256:(a + 1) * 256, :]
        new = []
        for i in range(4):
            new.append(jnp.maximum(sub[i], v))
            if i < 3:
                v = jnp.minimum(sub[i], v)
        sub = new

    def ce(x, y):
        return jnp.maximum(x, y), jnp.minimum(x, y)

    def merge22(a, b):
        e0, e1 = ce(a[0], b[0])
        o0, o1 = ce(a[1], b[1])
        m1, m2 = ce(o0, e1)
        return [e0, m1, m2, o1]

    def merge44(a, b):
        e = merge22([a[0], a[2]], [b[0], b[2]])
        o = merge22([a[1], a[3]], [b[1], b[3]])
        out = [e[0]]
        for i in range(3):
            hi, lo = ce(o[i], e[i + 1])
            out += [hi, lo]
        out.append(o[3])
        return out

    sk = [[sub[i][k * 64:(k + 1) * 64, :] for i in range(4)]
          for k in range(4)]
    m12 = merge44(sk[0], sk[1])
    m34 = merge44(sk[2], sk[3])
    c = [jnp.maximum(m12[i], m34[7 - i]) for i in range(8)]
    for i in range(4):
        c[i], c[i + 4] = ce(c[i], c[i + 4])
    for i in (0, 1, 4, 5):
        c[i], c[i + 2] = ce(c[i], c[i + 2])
    for i in (0, 2, 4):
        c[i], c[i + 1] = ce(c[i], c[i + 1])
    stacks = c[:nd]

    m = None
    for _ in range(kk):
        m = jnp.max(stacks[0], axis=0, keepdims=True)
        upd = stacks[0] == m
        stacks = ([jnp.where(upd, stacks[i + 1], stacks[i])
                   for i in range(nd - 1)]
                  + [jnp.where(upd, ninf, stacks[nd - 1])])
    tau = jnp.maximum(m.reshape(br, 1), 0.0)
    out_ref[...] = jnp.where(s >= tau, s, 0.0)


def kernel(features, W1, b1, W2, b2):
    n, d = features.shape
    emb = pl.pallas_call(
        _emb_body,
        out_shape=jax.ShapeDtypeStruct((n, d), jnp.float32),
    )(features, W1, b1.reshape(1, d), W2, b2.reshape(1, d))

    br = 256
    grid = (n // br,)
    out = pl.pallas_call(
        functools.partial(_sim_body, kk=K + 1),
        grid=grid,
        in_specs=[
            pl.BlockSpec((br, d), lambda i: (i, 0)),
            pl.BlockSpec((n, d), lambda i: (0, 0)),
        ],
        out_specs=pl.BlockSpec((br, n), lambda i: (i, 0)),
        out_shape=jax.ShapeDtypeStruct((n, n), jnp.float32),
    )(emb, emb)
    return out
